# baseline (speedup 1.0000x reference)
import functools

import jax
import jax.numpy as jnp
from jax import lax
from jax.experimental import pallas as pl
from jax.experimental.pallas import tpu as pltpu
from jax.experimental.pallas import tpu_sc as plsc

OCC_THRESHOLD = 0.5

R = 16384
P = 128
L = 16
NC = 2
NS = 16
NW = NC * NS
PER_W = R // NW
NB = PER_W // L
NG = P // L


def _body(occ_hbm, pos_hbm, am_hbm, buf0, bufs, posb, amb):
    w = lax.axis_index("s") * NC + lax.axis_index("c")
    base = w * PER_W
    lane = lax.iota(jnp.int32, L)
    zero = jnp.zeros((L,), jnp.int32)

    pltpu.sync_copy(
        occ_hbm.at[pl.ds(base * L, PER_W * L), pl.ds(0, 1)], buf0
    )

    def block_body(b, _):
        rows = (b * L + lane) * L
        m = plsc.load_gather(buf0, [rows, zero])
        am = zero
        for k in range(1, L):
            v = plsc.load_gather(buf0, [rows + k, zero])
            gt = v > m
            am = jnp.where(gt, k, am)
            m = jnp.maximum(v, m)
        qual = m >= OCC_THRESHOLD
        posb[pl.ds(b * L, L)] = qual.astype(jnp.int32)
        amb[pl.ds(b * L, L)] = am
        cnt = plsc.all_reduce_population_count(qual)[15]

        @pl.when(cnt < L)
        def _slow():
            pltpu.sync_copy(
                occ_hbm.at[pl.ds((base + b * L) * L, L * L), pl.ds(0, P)],
                bufs,
            )

            def ray_body(r, carry):
                posv, amv = carry

                def grp_body(g, ptv):
                    macc = bufs[r * L, pl.ds(g * L, L)]
                    for s in range(1, L):
                        macc = jnp.maximum(macc, bufs[r * L + s, pl.ds(g * L, L)])
                    hit = macc >= OCC_THRESHOLD
                    pg = jnp.min(jnp.where(hit, g * L + lane, P))
                    return jnp.minimum(ptv, pg)

                ptv = lax.fori_loop(0, NG, grp_body, P)
                pos = ptv < P
                pt = jnp.where(pos, ptv, 0)
                rv = jnp.full((L,), r * L, jnp.int32) + lane
                ptvv = jnp.full((L,), pt, jnp.int32)
                win = plsc.load_gather(bufs, [rv, ptvv])
                mx = jnp.max(win)
                am2 = jnp.min(jnp.where(win == mx, lane, L))
                sel = lane == r
                posv = jnp.where(sel, pos.astype(jnp.int32), posv)
                amv = jnp.where(sel, am2, amv)
                return posv, amv

            posv, amv = lax.fori_loop(0, L, ray_body, (zero, zero))
            posb[pl.ds(b * L, L)] = posv
            amb[pl.ds(b * L, L)] = amv

        return 0

    lax.fori_loop(0, NB, block_body, 0)
    pltpu.sync_copy(posb, pos_hbm.at[pl.ds(base, PER_W)])
    pltpu.sync_copy(amb, am_hbm.at[pl.ds(base, PER_W)])


@jax.jit
def _run(occ):
    mesh = plsc.VectorSubcoreMesh(core_axis_name="c", subcore_axis_name="s")
    f = pl.kernel(
        _body,
        out_type=(
            jax.ShapeDtypeStruct((R,), jnp.int32),
            jax.ShapeDtypeStruct((R,), jnp.int32),
        ),
        mesh=mesh,
        compiler_params=pltpu.CompilerParams(
            needs_layout_passes=False, use_tc_tiling_on_sc=False
        ),
        scratch_types=[
            pltpu.VMEM((PER_W * L, 1), jnp.float32),
            pltpu.VMEM((L * L, P), jnp.float32),
            pltpu.VMEM((PER_W,), jnp.int32),
            pltpu.VMEM((PER_W,), jnp.int32),
        ],
    )
    return f(occ.transpose(0, 2, 1).reshape(R * L, P))


def kernel(occ):
    pos, am = _run(occ)
    return (pos.astype(bool), am)


# speedup vs baseline: 1.8485x; 1.8485x over previous
import jax
import jax.numpy as jnp
from jax import lax
from jax.experimental import pallas as pl
from jax.experimental.pallas import tpu as pltpu
from jax.experimental.pallas import tpu_sc as plsc

OCC_THRESHOLD = 0.5

R = 16384
P = 128
L = 16
NG = P // L
NC = 2
NS = 16
NW = NC * NS
PER_W = R // NW
CH = 128
NCH = PER_W // CH
NBC = CH // L


def _body(occ_hbm, pos_hbm, am_hbm, buf0a, buf0b, bufs, posb, amb, sema, semb):
    w = lax.axis_index("s") * NC + lax.axis_index("c")
    base = w * PER_W
    lane = lax.iota(jnp.int32, L)
    zero = jnp.zeros((L,), jnp.int32)
    slabs = [buf0a, buf0b]
    sems = [sema, semb]

    def block_body(buf0, c, bb):
        b = c * NBC + bb
        rows = bb * L + lane
        m = plsc.load_gather(buf0, [rows, zero, zero])
        am = zero
        for k in range(1, L):
            kv = jnp.full((L,), k, jnp.int32)
            v = plsc.load_gather(buf0, [rows, kv, zero])
            gt = v > m
            am = jnp.where(gt, k, am)
            m = jnp.maximum(v, m)
        qual = m >= OCC_THRESHOLD
        posb[pl.ds(b * L, L)] = qual.astype(jnp.int32)
        amb[pl.ds(b * L, L)] = am
        cnt = plsc.all_reduce_population_count(qual)[15]

        @pl.when(cnt < L)
        def _slow():
            pltpu.sync_copy(
                occ_hbm.at[pl.ds(base + b * L, L), pl.ds(0, L), pl.ds(0, P)],
                bufs,
            )

            def ray_body(r, carry):
                posv, amv = carry

                def grp_body(g, ptv):
                    macc = bufs[r, 0, pl.ds(g * L, L)]
                    for s in range(1, L):
                        macc = jnp.maximum(macc, bufs[r, s, pl.ds(g * L, L)])
                    hit = macc >= OCC_THRESHOLD
                    pg = jnp.min(jnp.where(hit, g * L + lane, P))
                    return jnp.minimum(ptv, pg)

                ptv = lax.fori_loop(0, NG, grp_body, P)
                pos = ptv < P
                pt = jnp.where(pos, ptv, 0)
                rv = jnp.full((L,), r, jnp.int32)
                ptvv = jnp.full((L,), pt, jnp.int32)
                win = plsc.load_gather(bufs, [rv, lane, ptvv])
                mx = jnp.max(win)
                am2 = jnp.min(jnp.where(win == mx, lane, L))
                sel = lane == r
                posv = jnp.where(sel, pos.astype(jnp.int32), posv)
                amv = jnp.where(sel, am2, amv)
                return posv, amv

            posv, amv = lax.fori_loop(0, L, ray_body, (zero, zero))
            posb[pl.ds(b * L, L)] = posv
            amb[pl.ds(b * L, L)] = amv

        return 0

    def slab_copy(c, buf, sem):
        return pltpu.async_copy(
            occ_hbm.at[pl.ds(base + c * CH, CH), pl.ds(0, L), pl.ds(0, L)],
            buf,
            sem,
        )

    cp = slab_copy(0, slabs[0], sems[0])
    for c in range(NCH):
        if c + 1 < NCH:
            nxt = slab_copy(c + 1, slabs[(c + 1) % 2], sems[(c + 1) % 2])
        cp.wait()
        lax.fori_loop(0, NBC, lambda bb, x, _c=c: block_body(slabs[_c % 2], _c, bb), 0)
        if c + 1 < NCH:
            cp = nxt

    pltpu.sync_copy(posb, pos_hbm.at[pl.ds(base, PER_W)])
    pltpu.sync_copy(amb, am_hbm.at[pl.ds(base, PER_W)])


@jax.jit
def _run(occ):
    mesh = plsc.VectorSubcoreMesh(core_axis_name="c", subcore_axis_name="s")
    f = pl.kernel(
        _body,
        out_type=(
            jax.ShapeDtypeStruct((R,), jnp.int32),
            jax.ShapeDtypeStruct((R,), jnp.int32),
        ),
        mesh=mesh,
        compiler_params=pltpu.CompilerParams(
            needs_layout_passes=False, use_tc_tiling_on_sc=False
        ),
        scratch_types=[
            pltpu.VMEM((CH, L, L), jnp.float32),
            pltpu.VMEM((CH, L, L), jnp.float32),
            pltpu.VMEM((L, L, P), jnp.float32),
            pltpu.VMEM((PER_W,), jnp.int32),
            pltpu.VMEM((PER_W,), jnp.int32),
            pltpu.SemaphoreType.DMA,
            pltpu.SemaphoreType.DMA,
        ],
    )
    return f(occ.transpose(0, 2, 1))


def kernel(occ):
    pos, am = _run(occ)
    return (pos.astype(bool), am)
